# baseline (speedup 1.0000x reference)
import functools

import numpy as np
import jax
import jax.numpy as jnp
from jax import lax
from jax.experimental import pallas as pl
from jax.experimental.pallas import tpu as pltpu
from jax.experimental.pallas import tpu_sc as plsc

HEADS = 4
K_DIM = 512
KNN = 32
N_KEYS = 128
D_MODEL = 2048
N_TOKENS = 4096

TOK_BLK = 256
N_BLKS = N_TOKENS // TOK_BLK

_PREC = lax.Precision.DEFAULT
_PREC_SEL = lax.Precision.HIGHEST

_pairs = [(i, j) for i in range(KNN) for j in range(KNN) if (i + 1) * (j + 1) <= KNN]
_N_CAND = 128
_G1_np = np.zeros((KNN, _N_CAND), np.float32)
_G2_np = np.zeros((KNN, _N_CAND), np.float32)
_PAD_np = np.zeros((1, _N_CAND), np.float32)
for _p, (_i, _j) in enumerate(_pairs):
    _G1_np[_i, _p] = 1.0
    _G2_np[_j, _p] = 1.0
for _p in range(len(_pairs), _N_CAND):
    _PAD_np[0, _p] = -np.inf


def _top32(s, iota_n):
    B = s.shape[0]
    iota_k = lax.broadcasted_iota(jnp.int32, (B, KNN), 1)

    def body(k, carry):
        s, sc, idc = carry
        m = jnp.max(s, axis=1, keepdims=True)
        am = jnp.min(jnp.where(s == m, iota_n, s.shape[1]), axis=1, keepdims=True)
        s = jnp.where(iota_n == am, -jnp.inf, s)
        koh = iota_k == k
        sc = jnp.where(koh, m, sc)
        idc = jnp.where(koh, am, idc)
        return s, sc, idc

    init = (s, jnp.full((B, KNN), -jnp.inf, jnp.float32), jnp.zeros((B, KNN), jnp.int32))
    _, sc, idc = lax.fori_loop(0, KNN, body, init)
    return sc, idc


def _front_body(x_ref, wq_ref, bq_ref, k1t_ref, k2t_ref, ws_ref, bs_ref,
                g1_ref, g2_ref, pad_ref, gate_ref, idx_ref, wts_ref):
    B = TOK_BLK
    half = K_DIM // 2
    x = x_ref[...]
    q = jnp.dot(x, wq_ref[...], preferred_element_type=jnp.float32,
                precision=_PREC) + bq_ref[...]
    gate_ref[...] = jax.nn.silu(
        jnp.dot(x, ws_ref[...], preferred_element_type=jnp.float32,
                precision=_PREC) + bs_ref[...])

    iota_n = lax.broadcasted_iota(jnp.int32, (B, N_KEYS), 1)
    iota_c = lax.broadcasted_iota(jnp.int32, (B, _N_CAND), 1)
    g1 = g1_ref[...]
    g2 = g2_ref[...]
    pad = pad_ref[...]

    for h in range(HEADS):
        q1 = q[:, h * K_DIM:h * K_DIM + half]
        q2 = q[:, h * K_DIM + half:(h + 1) * K_DIM]
        s1 = jnp.dot(q1, k1t_ref[h], preferred_element_type=jnp.float32,
                     precision=_PREC)
        s2 = jnp.dot(q2, k2t_ref[h], preferred_element_type=jnp.float32,
                     precision=_PREC)
        sc1, id1 = _top32(s1, iota_n)
        sc2, id2 = _top32(s2, iota_n)
        c = (jnp.dot(sc1, g1, preferred_element_type=jnp.float32, precision=_PREC_SEL)
             + jnp.dot(sc2, g2, preferred_element_type=jnp.float32, precision=_PREC_SEL)
             + pad)
        icomb = (jnp.dot(id1.astype(jnp.float32), g1,
                         preferred_element_type=jnp.float32, precision=_PREC_SEL) * N_KEYS
                 + jnp.dot(id2.astype(jnp.float32), g2,
                           preferred_element_type=jnp.float32, precision=_PREC_SEL))

        iota_k = lax.broadcasted_iota(jnp.int32, (B, KNN), 1)

        def body2(k, carry):
            c, sc, idc = carry
            m = jnp.max(c, axis=1, keepdims=True)
            am = jnp.min(jnp.where(c == m, iota_c, _N_CAND), axis=1, keepdims=True)
            c = jnp.where(iota_c == am, -jnp.inf, c)
            iv = jnp.sum(jnp.where(iota_c == am, icomb, 0.0), axis=1, keepdims=True)
            koh = iota_k == k
            sc = jnp.where(koh, m, sc)
            idc = jnp.where(koh, iv, idc)
            return c, sc, idc

        init = (c, jnp.full((B, KNN), -jnp.inf, jnp.float32),
                jnp.zeros((B, KNN), jnp.float32))
        _, sc, idc = lax.fori_loop(0, KNN, body2, init)

        e = jnp.exp(sc - sc[:, 0:1])
        w = e / jnp.sum(e, axis=1, keepdims=True)
        idx_ref[:, h * KNN:(h + 1) * KNN] = idc.astype(jnp.int32)
        wts_ref[:, h * KNN:(h + 1) * KNN] = w


def _front_call(x, W_query, b_query, K1T, K2T, W_swilu, b_swilu, G1, G2, PAD):
    full = lambda shape: pl.BlockSpec(shape, lambda i: tuple(0 for _ in shape))
    return pl.pallas_call(
        _front_body,
        grid=(N_BLKS,),
        in_specs=[
            pl.BlockSpec((TOK_BLK, D_MODEL), lambda i: (i, 0)),
            full((D_MODEL, HEADS * K_DIM)),
            full((1, HEADS * K_DIM)),
            full((HEADS, K_DIM // 2, N_KEYS)),
            full((HEADS, K_DIM // 2, N_KEYS)),
            full((D_MODEL, D_MODEL)),
            full((1, D_MODEL)),
            full((KNN, _N_CAND)),
            full((KNN, _N_CAND)),
            full((1, _N_CAND)),
        ],
        out_specs=[
            pl.BlockSpec((TOK_BLK, D_MODEL), lambda i: (i, 0)),
            pl.BlockSpec((TOK_BLK, HEADS * KNN), lambda i: (i, 0)),
            pl.BlockSpec((TOK_BLK, HEADS * KNN), lambda i: (i, 0)),
        ],
        out_shape=[
            jax.ShapeDtypeStruct((N_TOKENS, D_MODEL), jnp.float32),
            jax.ShapeDtypeStruct((N_TOKENS, HEADS * KNN), jnp.int32),
            jax.ShapeDtypeStruct((N_TOKENS, HEADS * KNN), jnp.float32),
        ],
    )(x, W_query, b_query, K1T, K2T, W_swilu, b_swilu, G1, G2, PAD)


_N_WORKERS = 32
_TOK_PER_W = N_TOKENS // _N_WORKERS
_CHUNK = 8
_N_CHUNKS = (HEADS * KNN) // _CHUNK
_DCHUNKS = D_MODEL // 16


_D_UNROLL = 4
_NWORDS = D_MODEL // 2


def _bag_call(values, idx3, wspl3):
    mesh = plsc.VectorSubcoreMesh(core_axis_name="c", subcore_axis_name="s")

    @functools.partial(
        pl.kernel,
        out_type=jax.ShapeDtypeStruct((N_TOKENS, D_MODEL), jnp.float32),
        mesh=mesh,
        scratch_types=[
            pltpu.VMEM((_TOK_PER_W, _N_CHUNKS * _CHUNK), jnp.int32),
            pltpu.VMEM((_CHUNK, D_MODEL), jnp.float32),
            pltpu.VMEM((_CHUNK, D_MODEL), jnp.float32),
            pltpu.VMEM((_N_CHUNKS * _CHUNK, 16), jnp.float32),
            pltpu.VMEM((_N_CHUNKS * _CHUNK, 16), jnp.float32),
            pltpu.VMEM((D_MODEL,), jnp.float32),
            pltpu.VMEM((D_MODEL,), jnp.float32),
            pltpu.SemaphoreType.DMA,
            pltpu.SemaphoreType.DMA,
            pltpu.SemaphoreType.DMA,
            pltpu.SemaphoreType.DMA,
            pltpu.SemaphoreType.DMA,
            pltpu.SemaphoreType.DMA,
        ],
    )
    def bag(values_hbm, idx_hbm, wspl_hbm, out_hbm,
            idx_v, rows_a, rows_b, wspl_a, wspl_b, acc_a, acc_b,
            sem_ra, sem_rb, sem_wa, sem_wb, sem_oa, sem_ob):
        wid = lax.axis_index("s") * 2 + lax.axis_index("c")
        t0 = wid * _TOK_PER_W
        last = _TOK_PER_W - 1
        rbufs = (rows_a, rows_b)
        rsems = (sem_ra, sem_rb)
        wbufs = (wspl_a, wspl_b)
        wsems = (sem_wa, sem_wb)
        abufs = (acc_a, acc_b)
        osems = (sem_oa, sem_ob)

        pltpu.sync_copy(idx_hbm.at[pl.ds(t0, _TOK_PER_W)], idx_v)
        pltpu.async_copy(wspl_hbm.at[t0], wspl_a, sem_wa)
        pltpu.async_copy(wspl_hbm.at[t0 + 1], wspl_b, sem_wb)
        pltpu.async_copy(values_hbm.at[idx_v.at[0, pl.ds(0, _CHUNK)]], rows_a, sem_ra)

        def pair_body(p, _):
            for par in range(2):
                ti = 2 * p + par
                tok = t0 + ti
                wbuf = wbufs[par]
                acc = abufs[par]
                pltpu.make_async_copy(wspl_hbm.at[tok], wbuf, wsems[par]).wait()

                @pl.when(p > 0)
                def _():
                    pltpu.make_async_copy(
                        acc, out_hbm.at[tok - 2], osems[par]).wait()

                for c in range(_N_CHUNKS):
                    buf = rbufs[c % 2]
                    pltpu.make_async_copy(
                        values_hbm.at[idx_v.at[ti, pl.ds(c * _CHUNK, _CHUNK)]],
                        buf, rsems[c % 2]).wait()
                    if c + 1 < _N_CHUNKS:
                        pltpu.async_copy(
                            values_hbm.at[
                                idx_v.at[ti, pl.ds((c + 1) * _CHUNK, _CHUNK)]],
                            rbufs[(c + 1) % 2], rsems[(c + 1) % 2])
                    else:
                        tnext = jnp.minimum(ti + 1, last)
                        pltpu.async_copy(
                            values_hbm.at[idx_v.at[tnext, pl.ds(0, _CHUNK)]],
                            rbufs[0], rsems[0])
                    w = [wbuf.at[c * _CHUNK + r][pl.ds(0, 16)]
                         for r in range(_CHUNK)]

                    def d_body(dd):
                        sl = pl.ds(dd * 16, 16)
                        t = [buf.at[r][sl] * w[r] for r in range(_CHUNK)]
                        while len(t) > 1:
                            t = [t[i] + t[i + 1] for i in range(0, len(t), 2)]
                        s = t[0]
                        if c == 0:
                            acc[sl] = s
                        else:
                            plsc.addupdate(acc.at[sl], s)

                    plsc.parallel_loop(0, _DCHUNKS, 1, unroll=_D_UNROLL)(
                        lambda dd: d_body(dd))
                tpre = t0 + jnp.minimum(ti + 2, last)
                pltpu.async_copy(wspl_hbm.at[tpre], wbuf, wsems[par])
                pltpu.async_copy(acc, out_hbm.at[tok], osems[par])
            return 0

        lax.fori_loop(0, _TOK_PER_W // 2, pair_body, 0)
        pltpu.make_async_copy(
            wspl_hbm.at[t0 + last], wspl_a, sem_wa).wait()
        pltpu.make_async_copy(
            wspl_hbm.at[t0 + last], wspl_b, sem_wb).wait()
        pltpu.make_async_copy(
            values_hbm.at[idx_v.at[last, pl.ds(0, _CHUNK)]], rows_a, sem_ra).wait()
        pltpu.make_async_copy(
            acc_a, out_hbm.at[t0 + last - 1], sem_oa).wait()
        pltpu.make_async_copy(
            acc_b, out_hbm.at[t0 + last], sem_ob).wait()

    return bag(values, idx3, wspl3)


def _back_body(gate_ref, bag_ref, wv_ref, bv_ref, out_ref):
    y = gate_ref[...] * bag_ref[...]
    out_ref[...] = jnp.dot(y, wv_ref[...], preferred_element_type=jnp.float32,
                           precision=_PREC) + bv_ref[...]


def _back_call(gate, bag, W_vproj, b_vproj):
    return pl.pallas_call(
        _back_body,
        grid=(N_BLKS,),
        in_specs=[
            pl.BlockSpec((TOK_BLK, D_MODEL), lambda i: (i, 0)),
            pl.BlockSpec((TOK_BLK, D_MODEL), lambda i: (i, 0)),
            pl.BlockSpec((D_MODEL, D_MODEL), lambda i: (0, 0)),
            pl.BlockSpec((1, D_MODEL), lambda i: (0, 0)),
        ],
        out_specs=pl.BlockSpec((TOK_BLK, D_MODEL), lambda i: (i, 0)),
        out_shape=jax.ShapeDtypeStruct((N_TOKENS, D_MODEL), jnp.float32),
    )(gate, bag, W_vproj, b_vproj)


def kernel(input, W_query, b_query, keys, values, W_swilu, b_swilu, W_vproj, b_vproj):
    x = input.reshape(-1, D_MODEL)
    keys_r = keys.reshape(HEADS, 2, N_KEYS, K_DIM // 2)
    K1T = keys_r[:, 0].transpose(0, 2, 1)
    K2T = keys_r[:, 1].transpose(0, 2, 1)
    G1 = jnp.asarray(_G1_np)
    G2 = jnp.asarray(_G2_np)
    PAD = jnp.asarray(_PAD_np)
    gate, idx, wts = _front_call(
        x, W_query, b_query.reshape(1, -1), K1T, K2T,
        W_swilu, b_swilu.reshape(1, -1), G1, G2, PAD)
    wspl3 = jnp.broadcast_to(
        wts[..., None], (N_TOKENS, HEADS * KNN, 16))
    bag = _bag_call(values, idx, wspl3)
    return _back_call(gate, bag, W_vproj, b_vproj.reshape(1, -1))


# speedup vs baseline: 1.1729x; 1.1729x over previous
import functools

import numpy as np
import jax
import jax.numpy as jnp
from jax import lax
from jax.experimental import pallas as pl
from jax.experimental.pallas import tpu as pltpu
from jax.experimental.pallas import tpu_sc as plsc

HEADS = 4
K_DIM = 512
KNN = 32
N_KEYS = 128
D_MODEL = 2048
N_TOKENS = 4096

TOK_BLK = 256
N_BLKS = N_TOKENS // TOK_BLK

_PREC = lax.Precision.DEFAULT
_PREC_SEL = lax.Precision.HIGHEST

_pairs = [(i, j) for i in range(KNN) for j in range(KNN) if (i + 1) * (j + 1) <= KNN]
_N_CAND = 128
_G1_np = np.zeros((KNN, _N_CAND), np.float32)
_G2_np = np.zeros((KNN, _N_CAND), np.float32)
_PAD_np = np.zeros((1, _N_CAND), np.float32)
for _p, (_i, _j) in enumerate(_pairs):
    _G1_np[_i, _p] = 1.0
    _G2_np[_j, _p] = 1.0
for _p in range(len(_pairs), _N_CAND):
    _PAD_np[0, _p] = -np.inf


def _top32(s, iota_n):
    B = s.shape[0]
    iota_k = lax.broadcasted_iota(jnp.int32, (B, KNN), 1)

    def body(k, carry):
        s, sc, idc = carry
        m = jnp.max(s, axis=1, keepdims=True)
        am = jnp.min(jnp.where(s == m, iota_n, s.shape[1]), axis=1, keepdims=True)
        s = jnp.where(iota_n == am, -jnp.inf, s)
        koh = iota_k == k
        sc = jnp.where(koh, m, sc)
        idc = jnp.where(koh, am, idc)
        return s, sc, idc

    init = (s, jnp.full((B, KNN), -jnp.inf, jnp.float32), jnp.zeros((B, KNN), jnp.int32))
    _, sc, idc = lax.fori_loop(0, KNN, body, init)
    return sc, idc


def _front_body(x_ref, wq_ref, bq_ref, k1t_ref, k2t_ref, ws_ref, bs_ref,
                g1_ref, g2_ref, pad_ref, gate_ref, idx_ref, wts_ref):
    B = TOK_BLK
    half = K_DIM // 2
    x = x_ref[...]
    q = jnp.dot(x, wq_ref[...], preferred_element_type=jnp.float32,
                precision=_PREC) + bq_ref[...]
    gate_ref[...] = jax.nn.silu(
        jnp.dot(x, ws_ref[...], preferred_element_type=jnp.float32,
                precision=_PREC) + bs_ref[...])

    iota_n = lax.broadcasted_iota(jnp.int32, (B, N_KEYS), 1)
    iota_c = lax.broadcasted_iota(jnp.int32, (B, _N_CAND), 1)
    g1 = g1_ref[...]
    g2 = g2_ref[...]
    pad = pad_ref[...]

    for h in range(HEADS):
        q1 = q[:, h * K_DIM:h * K_DIM + half]
        q2 = q[:, h * K_DIM + half:(h + 1) * K_DIM]
        s1 = jnp.dot(q1, k1t_ref[h], preferred_element_type=jnp.float32,
                     precision=_PREC)
        s2 = jnp.dot(q2, k2t_ref[h], preferred_element_type=jnp.float32,
                     precision=_PREC)
        sc1, id1 = _top32(s1, iota_n)
        sc2, id2 = _top32(s2, iota_n)
        c = (jnp.dot(sc1, g1, preferred_element_type=jnp.float32, precision=_PREC_SEL)
             + jnp.dot(sc2, g2, preferred_element_type=jnp.float32, precision=_PREC_SEL)
             + pad)
        icomb = (jnp.dot(id1.astype(jnp.float32), g1,
                         preferred_element_type=jnp.float32, precision=_PREC_SEL) * N_KEYS
                 + jnp.dot(id2.astype(jnp.float32), g2,
                           preferred_element_type=jnp.float32, precision=_PREC_SEL))

        iota_k = lax.broadcasted_iota(jnp.int32, (B, KNN), 1)

        def body2(k, carry):
            c, sc, idc = carry
            m = jnp.max(c, axis=1, keepdims=True)
            am = jnp.min(jnp.where(c == m, iota_c, _N_CAND), axis=1, keepdims=True)
            c = jnp.where(iota_c == am, -jnp.inf, c)
            iv = jnp.sum(jnp.where(iota_c == am, icomb, 0.0), axis=1, keepdims=True)
            koh = iota_k == k
            sc = jnp.where(koh, m, sc)
            idc = jnp.where(koh, iv, idc)
            return c, sc, idc

        init = (c, jnp.full((B, KNN), -jnp.inf, jnp.float32),
                jnp.zeros((B, KNN), jnp.float32))
        _, sc, idc = lax.fori_loop(0, KNN, body2, init)

        e = jnp.exp(sc - sc[:, 0:1])
        w = e / jnp.sum(e, axis=1, keepdims=True)
        idx_ref[:, h * KNN:(h + 1) * KNN] = idc.astype(jnp.int32)
        wts_ref[:, h * KNN:(h + 1) * KNN] = w


def _front_call(x, W_query, b_query, K1T, K2T, W_swilu, b_swilu, G1, G2, PAD):
    full = lambda shape: pl.BlockSpec(shape, lambda i: tuple(0 for _ in shape))
    return pl.pallas_call(
        _front_body,
        grid=(N_BLKS,),
        in_specs=[
            pl.BlockSpec((TOK_BLK, D_MODEL), lambda i: (i, 0)),
            full((D_MODEL, HEADS * K_DIM)),
            full((1, HEADS * K_DIM)),
            full((HEADS, K_DIM // 2, N_KEYS)),
            full((HEADS, K_DIM // 2, N_KEYS)),
            full((D_MODEL, D_MODEL)),
            full((1, D_MODEL)),
            full((KNN, _N_CAND)),
            full((KNN, _N_CAND)),
            full((1, _N_CAND)),
        ],
        out_specs=[
            pl.BlockSpec((TOK_BLK, D_MODEL), lambda i: (i, 0)),
            pl.BlockSpec((TOK_BLK, HEADS * KNN), lambda i: (i, 0)),
            pl.BlockSpec((TOK_BLK, HEADS * KNN), lambda i: (i, 0)),
        ],
        out_shape=[
            jax.ShapeDtypeStruct((N_TOKENS, D_MODEL), jnp.float32),
            jax.ShapeDtypeStruct((N_TOKENS, HEADS * KNN), jnp.int32),
            jax.ShapeDtypeStruct((N_TOKENS, HEADS * KNN), jnp.float32),
        ],
    )(x, W_query, b_query, K1T, K2T, W_swilu, b_swilu, G1, G2, PAD)


_N_WORKERS = 32
_TOK_PER_W = N_TOKENS // _N_WORKERS
_CHUNK = 16
_N_CHUNKS = (HEADS * KNN) // _CHUNK
_DCHUNKS = D_MODEL // 16


_D_UNROLL = 4
_NWORDS = D_MODEL // 2


def _bag_call(values, idx3, wspl3):
    mesh = plsc.VectorSubcoreMesh(core_axis_name="c", subcore_axis_name="s")

    @functools.partial(
        pl.kernel,
        out_type=jax.ShapeDtypeStruct((N_TOKENS, D_MODEL), jnp.float32),
        mesh=mesh,
        scratch_types=[
            pltpu.VMEM((_TOK_PER_W, _N_CHUNKS * _CHUNK), jnp.int32),
            pltpu.VMEM((_CHUNK, D_MODEL), jnp.float32),
            pltpu.VMEM((_CHUNK, D_MODEL), jnp.float32),
            pltpu.VMEM((_N_CHUNKS * _CHUNK * 16,), jnp.float32),
            pltpu.VMEM((_N_CHUNKS * _CHUNK * 16,), jnp.float32),
            pltpu.VMEM((D_MODEL,), jnp.float32),
            pltpu.VMEM((D_MODEL,), jnp.float32),
            pltpu.SemaphoreType.DMA,
            pltpu.SemaphoreType.DMA,
            pltpu.SemaphoreType.DMA,
            pltpu.SemaphoreType.DMA,
            pltpu.SemaphoreType.DMA,
            pltpu.SemaphoreType.DMA,
        ],
    )
    def bag(values_hbm, idx_hbm, wspl_hbm, out_hbm,
            idx_v, rows_a, rows_b, wspl_a, wspl_b, acc_a, acc_b,
            sem_ra, sem_rb, sem_wa, sem_wb, sem_oa, sem_ob):
        wid = lax.axis_index("s") * 2 + lax.axis_index("c")
        t0 = wid * _TOK_PER_W
        last = _TOK_PER_W - 1
        rbufs = (rows_a, rows_b)
        rsems = (sem_ra, sem_rb)
        wbufs = (wspl_a, wspl_b)
        wsems = (sem_wa, sem_wb)
        abufs = (acc_a, acc_b)
        osems = (sem_oa, sem_ob)

        pltpu.sync_copy(idx_hbm.at[pl.ds(t0, _TOK_PER_W)], idx_v)
        pltpu.async_copy(wspl_hbm.at[t0], wspl_a, sem_wa)
        pltpu.async_copy(wspl_hbm.at[t0 + 1], wspl_b, sem_wb)
        pltpu.async_copy(values_hbm.at[idx_v.at[0, pl.ds(0, _CHUNK)]], rows_a, sem_ra)

        def pair_body(p, _):
            for par in range(2):
                ti = 2 * p + par
                tok = t0 + ti
                wbuf = wbufs[par]
                acc = abufs[par]
                pltpu.make_async_copy(wspl_hbm.at[tok], wbuf, wsems[par]).wait()

                @pl.when(p > 0)
                def _():
                    pltpu.make_async_copy(
                        acc, out_hbm.at[tok - 2], osems[par]).wait()

                for c in range(_N_CHUNKS):
                    buf = rbufs[c % 2]
                    pltpu.make_async_copy(
                        values_hbm.at[idx_v.at[ti, pl.ds(c * _CHUNK, _CHUNK)]],
                        buf, rsems[c % 2]).wait()
                    if c + 1 < _N_CHUNKS:
                        pltpu.async_copy(
                            values_hbm.at[
                                idx_v.at[ti, pl.ds((c + 1) * _CHUNK, _CHUNK)]],
                            rbufs[(c + 1) % 2], rsems[(c + 1) % 2])
                    else:
                        tnext = jnp.minimum(ti + 1, last)
                        pltpu.async_copy(
                            values_hbm.at[idx_v.at[tnext, pl.ds(0, _CHUNK)]],
                            rbufs[0], rsems[0])
                    w = [wbuf[pl.ds((c * _CHUNK + r) * 16, 16)]
                         for r in range(_CHUNK)]

                    def d_body(dd):
                        sl = pl.ds(dd * 16, 16)
                        t = [buf.at[r][sl] * w[r] for r in range(_CHUNK)]
                        while len(t) > 1:
                            t = [t[i] + t[i + 1] for i in range(0, len(t), 2)]
                        s = t[0]
                        if c == 0:
                            acc[sl] = s
                        else:
                            plsc.addupdate(acc.at[sl], s)

                    plsc.parallel_loop(0, _DCHUNKS, 1, unroll=_D_UNROLL)(
                        lambda dd: d_body(dd))
                tpre = t0 + jnp.minimum(ti + 2, last)
                pltpu.async_copy(wspl_hbm.at[tpre], wbuf, wsems[par])
                pltpu.async_copy(acc, out_hbm.at[tok], osems[par])
            return 0

        lax.fori_loop(0, _TOK_PER_W // 2, pair_body, 0)
        pltpu.make_async_copy(
            wspl_hbm.at[t0 + last], wspl_a, sem_wa).wait()
        pltpu.make_async_copy(
            wspl_hbm.at[t0 + last], wspl_b, sem_wb).wait()
        pltpu.make_async_copy(
            values_hbm.at[idx_v.at[last, pl.ds(0, _CHUNK)]], rows_a, sem_ra).wait()
        pltpu.make_async_copy(
            acc_a, out_hbm.at[t0 + last - 1], sem_oa).wait()
        pltpu.make_async_copy(
            acc_b, out_hbm.at[t0 + last], sem_ob).wait()

    return bag(values, idx3, wspl3)


def _back_body(gate_ref, bag_ref, wv_ref, bv_ref, out_ref):
    y = gate_ref[...] * bag_ref[...]
    out_ref[...] = jnp.dot(y, wv_ref[...], preferred_element_type=jnp.float32,
                           precision=_PREC) + bv_ref[...]


def _back_call(gate, bag, W_vproj, b_vproj):
    return pl.pallas_call(
        _back_body,
        grid=(N_BLKS,),
        in_specs=[
            pl.BlockSpec((TOK_BLK, D_MODEL), lambda i: (i, 0)),
            pl.BlockSpec((TOK_BLK, D_MODEL), lambda i: (i, 0)),
            pl.BlockSpec((D_MODEL, D_MODEL), lambda i: (0, 0)),
            pl.BlockSpec((1, D_MODEL), lambda i: (0, 0)),
        ],
        out_specs=pl.BlockSpec((TOK_BLK, D_MODEL), lambda i: (i, 0)),
        out_shape=jax.ShapeDtypeStruct((N_TOKENS, D_MODEL), jnp.float32),
    )(gate, bag, W_vproj, b_vproj)


def kernel(input, W_query, b_query, keys, values, W_swilu, b_swilu, W_vproj, b_vproj):
    x = input.reshape(-1, D_MODEL)
    keys_r = keys.reshape(HEADS, 2, N_KEYS, K_DIM // 2)
    K1T = keys_r[:, 0].transpose(0, 2, 1)
    K2T = keys_r[:, 1].transpose(0, 2, 1)
    G1 = jnp.asarray(_G1_np)
    G2 = jnp.asarray(_G2_np)
    PAD = jnp.asarray(_PAD_np)
    gate, idx, wts = _front_call(
        x, W_query, b_query.reshape(1, -1), K1T, K2T,
        W_swilu, b_swilu.reshape(1, -1), G1, G2, PAD)
    wspl3 = jnp.broadcast_to(
        wts[..., None], (N_TOKENS, HEADS * KNN, 16)
    ).reshape(N_TOKENS, HEADS * KNN * 16)
    bag = _bag_call(values, idx, wspl3)
    return _back_call(gate, bag, W_vproj, b_vproj.reshape(1, -1))


# speedup vs baseline: 1.5263x; 1.3012x over previous
import functools

import numpy as np
import jax
import jax.numpy as jnp
from jax import lax
from jax.experimental import pallas as pl
from jax.experimental.pallas import tpu as pltpu
from jax.experimental.pallas import tpu_sc as plsc

HEADS = 4
K_DIM = 512
KNN = 32
N_KEYS = 128
D_MODEL = 2048
N_TOKENS = 4096

TOK_BLK = 256
N_BLKS = N_TOKENS // TOK_BLK

_PREC = lax.Precision.DEFAULT
_PREC_SEL = lax.Precision.HIGHEST

_pairs = [(i, j) for i in range(KNN) for j in range(KNN) if (i + 1) * (j + 1) <= KNN]
_N_CAND = 128
_G1_np = np.zeros((KNN, _N_CAND), np.float32)
_G2_np = np.zeros((KNN, _N_CAND), np.float32)
_PAD_np = np.zeros((1, _N_CAND), np.float32)
for _p, (_i, _j) in enumerate(_pairs):
    _G1_np[_i, _p] = 1.0
    _G2_np[_j, _p] = 1.0
for _p in range(len(_pairs), _N_CAND):
    _PAD_np[0, _p] = -np.inf


def _top32(s, iota_n):
    B = s.shape[0]
    iota_k = lax.broadcasted_iota(jnp.int32, (B, KNN), 1)

    def body(k, carry):
        s, sc, idc = carry
        m = jnp.max(s, axis=1, keepdims=True)
        am = jnp.min(jnp.where(s == m, iota_n, s.shape[1]), axis=1, keepdims=True)
        s = jnp.where(iota_n == am, -jnp.inf, s)
        koh = iota_k == k
        sc = jnp.where(koh, m, sc)
        idc = jnp.where(koh, am, idc)
        return s, sc, idc

    init = (s, jnp.full((B, KNN), -jnp.inf, jnp.float32), jnp.zeros((B, KNN), jnp.int32))
    _, sc, idc = lax.fori_loop(0, KNN, body, init)
    return sc, idc


def _front_body(x_ref, wq_ref, bq_ref, k1t_ref, k2t_ref, ws_ref, bs_ref,
                g1_ref, g2_ref, pad_ref, gate_ref, idx_ref, wts_ref):
    B = TOK_BLK
    half = K_DIM // 2
    x = x_ref[...]
    q = jnp.dot(x, wq_ref[...], preferred_element_type=jnp.float32,
                precision=_PREC) + bq_ref[...]
    gate_ref[...] = jax.nn.silu(
        jnp.dot(x, ws_ref[...], preferred_element_type=jnp.float32,
                precision=_PREC) + bs_ref[...])

    iota_n = lax.broadcasted_iota(jnp.int32, (B, N_KEYS), 1)
    iota_c = lax.broadcasted_iota(jnp.int32, (B, _N_CAND), 1)
    g1 = g1_ref[...]
    g2 = g2_ref[...]
    pad = pad_ref[...]

    for h in range(HEADS):
        q1 = q[:, h * K_DIM:h * K_DIM + half]
        q2 = q[:, h * K_DIM + half:(h + 1) * K_DIM]
        s1 = jnp.dot(q1, k1t_ref[h], preferred_element_type=jnp.float32,
                     precision=_PREC)
        s2 = jnp.dot(q2, k2t_ref[h], preferred_element_type=jnp.float32,
                     precision=_PREC)
        sc1, id1 = _top32(s1, iota_n)
        sc2, id2 = _top32(s2, iota_n)
        c = (jnp.dot(sc1, g1, preferred_element_type=jnp.float32, precision=_PREC_SEL)
             + jnp.dot(sc2, g2, preferred_element_type=jnp.float32, precision=_PREC_SEL)
             + pad)
        icomb = (jnp.dot(id1.astype(jnp.float32), g1,
                         preferred_element_type=jnp.float32, precision=_PREC_SEL) * N_KEYS
                 + jnp.dot(id2.astype(jnp.float32), g2,
                           preferred_element_type=jnp.float32, precision=_PREC_SEL))

        iota_k = lax.broadcasted_iota(jnp.int32, (B, KNN), 1)

        def body2(k, carry):
            c, sc, idc = carry
            m = jnp.max(c, axis=1, keepdims=True)
            am = jnp.min(jnp.where(c == m, iota_c, _N_CAND), axis=1, keepdims=True)
            c = jnp.where(iota_c == am, -jnp.inf, c)
            iv = jnp.sum(jnp.where(iota_c == am, icomb, 0.0), axis=1, keepdims=True)
            koh = iota_k == k
            sc = jnp.where(koh, m, sc)
            idc = jnp.where(koh, iv, idc)
            return c, sc, idc

        init = (c, jnp.full((B, KNN), -jnp.inf, jnp.float32),
                jnp.zeros((B, KNN), jnp.float32))
        _, sc, idc = lax.fori_loop(0, KNN, body2, init)

        e = jnp.exp(sc - sc[:, 0:1])
        w = e / jnp.sum(e, axis=1, keepdims=True)
        idx_ref[:, h * KNN:(h + 1) * KNN] = idc.astype(jnp.int32)
        wts_ref[:, h * KNN:(h + 1) * KNN] = w


def _front_call(x, W_query, b_query, K1T, K2T, W_swilu, b_swilu, G1, G2, PAD):
    ntok = x.shape[0]
    full = lambda shape: pl.BlockSpec(shape, lambda i: tuple(0 for _ in shape))
    return pl.pallas_call(
        _front_body,
        grid=(ntok // TOK_BLK,),
        in_specs=[
            pl.BlockSpec((TOK_BLK, D_MODEL), lambda i: (i, 0)),
            full((D_MODEL, HEADS * K_DIM)),
            full((1, HEADS * K_DIM)),
            full((HEADS, K_DIM // 2, N_KEYS)),
            full((HEADS, K_DIM // 2, N_KEYS)),
            full((D_MODEL, D_MODEL)),
            full((1, D_MODEL)),
            full((KNN, _N_CAND)),
            full((KNN, _N_CAND)),
            full((1, _N_CAND)),
        ],
        out_specs=[
            pl.BlockSpec((TOK_BLK, D_MODEL), lambda i: (i, 0)),
            pl.BlockSpec((TOK_BLK, HEADS * KNN), lambda i: (i, 0)),
            pl.BlockSpec((TOK_BLK, HEADS * KNN), lambda i: (i, 0)),
        ],
        out_shape=[
            jax.ShapeDtypeStruct((ntok, D_MODEL), jnp.float32),
            jax.ShapeDtypeStruct((ntok, HEADS * KNN), jnp.int32),
            jax.ShapeDtypeStruct((ntok, HEADS * KNN), jnp.float32),
        ],
    )(x, W_query, b_query, K1T, K2T, W_swilu, b_swilu, G1, G2, PAD)


_N_WORKERS = 32
_TOK_PER_W = N_TOKENS // _N_WORKERS
_CHUNK = 16
_N_CHUNKS = (HEADS * KNN) // _CHUNK
_DCHUNKS = D_MODEL // 16


_D_UNROLL = 4
_N_HALVES = 2
_NWORDS = D_MODEL // 2


def _bag_call(values, idx3, wspl3):
    ntok = idx3.shape[0]
    tok_per_w = ntok // _N_WORKERS
    mesh = plsc.VectorSubcoreMesh(core_axis_name="c", subcore_axis_name="s")

    @functools.partial(
        pl.kernel,
        out_type=jax.ShapeDtypeStruct((ntok, D_MODEL), jnp.float32),
        mesh=mesh,
        scratch_types=[
            pltpu.VMEM((tok_per_w, _N_CHUNKS * _CHUNK), jnp.int32),
            pltpu.VMEM((_CHUNK, D_MODEL), jnp.float32),
            pltpu.VMEM((_CHUNK, D_MODEL), jnp.float32),
            pltpu.VMEM((_N_CHUNKS * _CHUNK * 16,), jnp.float32),
            pltpu.VMEM((_N_CHUNKS * _CHUNK * 16,), jnp.float32),
            pltpu.VMEM((D_MODEL,), jnp.float32),
            pltpu.VMEM((D_MODEL,), jnp.float32),
            pltpu.SemaphoreType.DMA,
            pltpu.SemaphoreType.DMA,
            pltpu.SemaphoreType.DMA,
            pltpu.SemaphoreType.DMA,
            pltpu.SemaphoreType.DMA,
            pltpu.SemaphoreType.DMA,
        ],
    )
    def bag(values_hbm, idx_hbm, wspl_hbm, out_hbm,
            idx_v, rows_a, rows_b, wspl_a, wspl_b, acc_a, acc_b,
            sem_ra, sem_rb, sem_wa, sem_wb, sem_oa, sem_ob):
        wid = lax.axis_index("s") * 2 + lax.axis_index("c")
        t0 = wid * tok_per_w
        last = tok_per_w - 1
        rbufs = (rows_a, rows_b)
        rsems = (sem_ra, sem_rb)
        wbufs = (wspl_a, wspl_b)
        wsems = (sem_wa, sem_wb)
        abufs = (acc_a, acc_b)
        osems = (sem_oa, sem_ob)

        pltpu.sync_copy(idx_hbm.at[pl.ds(t0, tok_per_w)], idx_v)
        pltpu.async_copy(wspl_hbm.at[t0], wspl_a, sem_wa)
        pltpu.async_copy(wspl_hbm.at[t0 + 1], wspl_b, sem_wb)
        pltpu.async_copy(values_hbm.at[idx_v.at[0, pl.ds(0, _CHUNK)]], rows_a, sem_ra)

        def pair_body(p, _):
            for par in range(2):
                ti = 2 * p + par
                tok = t0 + ti
                wbuf = wbufs[par]
                acc = abufs[par]
                pltpu.make_async_copy(wspl_hbm.at[tok], wbuf, wsems[par]).wait()

                @pl.when(p > 0)
                def _():
                    pltpu.make_async_copy(
                        acc, out_hbm.at[tok - 2], osems[par]).wait()

                for c in range(_N_CHUNKS):
                    buf = rbufs[c % 2]
                    pltpu.make_async_copy(
                        values_hbm.at[idx_v.at[ti, pl.ds(c * _CHUNK, _CHUNK)]],
                        buf, rsems[c % 2]).wait()
                    if c + 1 < _N_CHUNKS:
                        pltpu.async_copy(
                            values_hbm.at[
                                idx_v.at[ti, pl.ds((c + 1) * _CHUNK, _CHUNK)]],
                            rbufs[(c + 1) % 2], rsems[(c + 1) % 2])
                    else:
                        tnext = jnp.minimum(ti + 1, last)
                        pltpu.async_copy(
                            values_hbm.at[idx_v.at[tnext, pl.ds(0, _CHUNK)]],
                            rbufs[0], rsems[0])
                    w = [wbuf[pl.ds((c * _CHUNK + r) * 16, 16)]
                         for r in range(_CHUNK)]

                    def d_body(dd):
                        sl = pl.ds(dd * 16, 16)
                        t = [buf.at[r][sl] * w[r] for r in range(_CHUNK)]
                        while len(t) > 1:
                            t = [t[i] + t[i + 1] for i in range(0, len(t), 2)]
                        s = t[0]
                        if c == 0:
                            acc[sl] = s
                        else:
                            plsc.addupdate(acc.at[sl], s)

                    plsc.parallel_loop(0, _DCHUNKS, 1, unroll=_D_UNROLL)(
                        lambda dd: d_body(dd))
                tpre = t0 + jnp.minimum(ti + 2, last)
                pltpu.async_copy(wspl_hbm.at[tpre], wbuf, wsems[par])
                pltpu.async_copy(acc, out_hbm.at[tok], osems[par])
            return 0

        lax.fori_loop(0, tok_per_w // 2, pair_body, 0)
        pltpu.make_async_copy(
            wspl_hbm.at[t0 + last], wspl_a, sem_wa).wait()
        pltpu.make_async_copy(
            wspl_hbm.at[t0 + last], wspl_b, sem_wb).wait()
        pltpu.make_async_copy(
            values_hbm.at[idx_v.at[last, pl.ds(0, _CHUNK)]], rows_a, sem_ra).wait()
        pltpu.make_async_copy(
            acc_a, out_hbm.at[t0 + last - 1], sem_oa).wait()
        pltpu.make_async_copy(
            acc_b, out_hbm.at[t0 + last], sem_ob).wait()

    return bag(values, idx3, wspl3)


def _back_body(gate_ref, bag_ref, wv_ref, bv_ref, out_ref):
    y = gate_ref[...] * bag_ref[...]
    out_ref[...] = jnp.dot(y, wv_ref[...], preferred_element_type=jnp.float32,
                           precision=_PREC) + bv_ref[...]


def _back_call(gate, bag, W_vproj, b_vproj):
    ntok = gate.shape[0]
    return pl.pallas_call(
        _back_body,
        grid=(ntok // TOK_BLK,),
        in_specs=[
            pl.BlockSpec((TOK_BLK, D_MODEL), lambda i: (i, 0)),
            pl.BlockSpec((TOK_BLK, D_MODEL), lambda i: (i, 0)),
            pl.BlockSpec((D_MODEL, D_MODEL), lambda i: (0, 0)),
            pl.BlockSpec((1, D_MODEL), lambda i: (0, 0)),
        ],
        out_specs=pl.BlockSpec((TOK_BLK, D_MODEL), lambda i: (i, 0)),
        out_shape=jax.ShapeDtypeStruct((ntok, D_MODEL), jnp.float32),
    )(gate, bag, W_vproj, b_vproj)


def kernel(input, W_query, b_query, keys, values, W_swilu, b_swilu, W_vproj, b_vproj):
    x = input.reshape(-1, D_MODEL)
    keys_r = keys.reshape(HEADS, 2, N_KEYS, K_DIM // 2)
    K1T = keys_r[:, 0].transpose(0, 2, 1)
    K2T = keys_r[:, 1].transpose(0, 2, 1)
    G1 = jnp.asarray(_G1_np)
    G2 = jnp.asarray(_G2_np)
    PAD = jnp.asarray(_PAD_np)
    halves = []
    nh = N_TOKENS // _N_HALVES
    for hh in range(_N_HALVES):
        xh = x[hh * nh:(hh + 1) * nh]
        gate, idx, wts = _front_call(
            xh, W_query, b_query.reshape(1, -1), K1T, K2T,
            W_swilu, b_swilu.reshape(1, -1), G1, G2, PAD)
        wspl3 = jnp.broadcast_to(
            wts[..., None], (nh, HEADS * KNN, 16)
        ).reshape(nh, HEADS * KNN * 16)
        bag = _bag_call(values, idx, wspl3)
        halves.append(_back_call(gate, bag, W_vproj, b_vproj.reshape(1, -1)))
    return jnp.concatenate(halves, axis=0)


# speedup vs baseline: 1.7838x; 1.1688x over previous
import functools

import numpy as np
import jax
import jax.numpy as jnp
from jax import lax
from jax.experimental import pallas as pl
from jax.experimental.pallas import tpu as pltpu
from jax.experimental.pallas import tpu_sc as plsc

HEADS = 4
K_DIM = 512
KNN = 32
N_KEYS = 128
D_MODEL = 2048
N_TOKENS = 4096

TOK_BLK = 256
N_BLKS = N_TOKENS // TOK_BLK

_PREC = lax.Precision.DEFAULT
_PREC_SEL = lax.Precision.HIGHEST

_pairs = [(i, j) for i in range(KNN) for j in range(KNN) if (i + 1) * (j + 1) <= KNN]
_N_CAND = 128
_G1_np = np.zeros((KNN, _N_CAND), np.float32)
_G2_np = np.zeros((KNN, _N_CAND), np.float32)
_PAD_np = np.zeros((1, _N_CAND), np.float32)
for _p, (_i, _j) in enumerate(_pairs):
    _G1_np[_i, _p] = 1.0
    _G2_np[_j, _p] = 1.0
for _p in range(len(_pairs), _N_CAND):
    _PAD_np[0, _p] = -np.inf


def _top32(s, iota_n):
    B = s.shape[0]
    iota_k = lax.broadcasted_iota(jnp.int32, (B, KNN), 1)

    def body(k, carry):
        s, sc, idc = carry
        m = jnp.max(s, axis=1, keepdims=True)
        am = jnp.min(jnp.where(s == m, iota_n, s.shape[1]), axis=1, keepdims=True)
        s = jnp.where(iota_n == am, -jnp.inf, s)
        koh = iota_k == k
        sc = jnp.where(koh, m, sc)
        idc = jnp.where(koh, am, idc)
        return s, sc, idc

    init = (s, jnp.full((B, KNN), -jnp.inf, jnp.float32), jnp.zeros((B, KNN), jnp.int32))
    _, sc, idc = lax.fori_loop(0, KNN, body, init)
    return sc, idc


def _front_body(x_ref, wq_ref, bq_ref, k1t_ref, k2t_ref, ws_ref, bs_ref,
                g1_ref, g2_ref, pad_ref, gate_ref, idx_ref, wts_ref):
    B = TOK_BLK
    half = K_DIM // 2
    x = x_ref[...]
    q = jnp.dot(x, wq_ref[...], preferred_element_type=jnp.float32,
                precision=_PREC) + bq_ref[...]
    gate_ref[...] = jax.nn.silu(
        jnp.dot(x, ws_ref[...], preferred_element_type=jnp.float32,
                precision=_PREC) + bs_ref[...])

    iota_n = lax.broadcasted_iota(jnp.int32, (B, N_KEYS), 1)
    iota_c = lax.broadcasted_iota(jnp.int32, (B, _N_CAND), 1)
    g1 = g1_ref[...]
    g2 = g2_ref[...]
    pad = pad_ref[...]

    for h in range(HEADS):
        q1 = q[:, h * K_DIM:h * K_DIM + half]
        q2 = q[:, h * K_DIM + half:(h + 1) * K_DIM]
        s1 = jnp.dot(q1, k1t_ref[h], preferred_element_type=jnp.float32,
                     precision=_PREC)
        s2 = jnp.dot(q2, k2t_ref[h], preferred_element_type=jnp.float32,
                     precision=_PREC)
        sc1, id1 = _top32(s1, iota_n)
        sc2, id2 = _top32(s2, iota_n)
        c = (jnp.dot(sc1, g1, preferred_element_type=jnp.float32, precision=_PREC_SEL)
             + jnp.dot(sc2, g2, preferred_element_type=jnp.float32, precision=_PREC_SEL)
             + pad)
        icomb = (jnp.dot(id1.astype(jnp.float32), g1,
                         preferred_element_type=jnp.float32, precision=_PREC_SEL) * N_KEYS
                 + jnp.dot(id2.astype(jnp.float32), g2,
                           preferred_element_type=jnp.float32, precision=_PREC_SEL))

        iota_k = lax.broadcasted_iota(jnp.int32, (B, KNN), 1)

        def body2(k, carry):
            c, sc, idc = carry
            m = jnp.max(c, axis=1, keepdims=True)
            am = jnp.min(jnp.where(c == m, iota_c, _N_CAND), axis=1, keepdims=True)
            c = jnp.where(iota_c == am, -jnp.inf, c)
            iv = jnp.sum(jnp.where(iota_c == am, icomb, 0.0), axis=1, keepdims=True)
            koh = iota_k == k
            sc = jnp.where(koh, m, sc)
            idc = jnp.where(koh, iv, idc)
            return c, sc, idc

        init = (c, jnp.full((B, KNN), -jnp.inf, jnp.float32),
                jnp.zeros((B, KNN), jnp.float32))
        _, sc, idc = lax.fori_loop(0, KNN, body2, init)

        e = jnp.exp(sc - sc[:, 0:1])
        w = e / jnp.sum(e, axis=1, keepdims=True)
        idx_ref[:, h * KNN:(h + 1) * KNN] = idc.astype(jnp.int32)
        wts_ref[:, h * KNN:(h + 1) * KNN] = w


def _front_call(x, W_query, b_query, K1T, K2T, W_swilu, b_swilu, G1, G2, PAD):
    ntok = x.shape[0]
    full = lambda shape: pl.BlockSpec(shape, lambda i: tuple(0 for _ in shape))
    return pl.pallas_call(
        _front_body,
        grid=(ntok // TOK_BLK,),
        in_specs=[
            pl.BlockSpec((TOK_BLK, D_MODEL), lambda i: (i, 0)),
            full((D_MODEL, HEADS * K_DIM)),
            full((1, HEADS * K_DIM)),
            full((HEADS, K_DIM // 2, N_KEYS)),
            full((HEADS, K_DIM // 2, N_KEYS)),
            full((D_MODEL, D_MODEL)),
            full((1, D_MODEL)),
            full((KNN, _N_CAND)),
            full((KNN, _N_CAND)),
            full((1, _N_CAND)),
        ],
        out_specs=[
            pl.BlockSpec((TOK_BLK, D_MODEL), lambda i: (i, 0)),
            pl.BlockSpec((TOK_BLK, HEADS * KNN), lambda i: (i, 0)),
            pl.BlockSpec((TOK_BLK, HEADS * KNN), lambda i: (i, 0)),
        ],
        out_shape=[
            jax.ShapeDtypeStruct((ntok, D_MODEL), jnp.float32),
            jax.ShapeDtypeStruct((ntok, HEADS * KNN), jnp.int32),
            jax.ShapeDtypeStruct((ntok, HEADS * KNN), jnp.float32),
        ],
    )(x, W_query, b_query, K1T, K2T, W_swilu, b_swilu, G1, G2, PAD)


_N_WORKERS = 32
_TOK_PER_W = N_TOKENS // _N_WORKERS
_CHUNK = 16
_N_CHUNKS = (HEADS * KNN) // _CHUNK
_DCHUNKS = D_MODEL // 16


_D_UNROLL = 4
_N_HALVES = 4
_NWORDS = D_MODEL // 2


def _bag_call(values, idx3, wspl3):
    ntok = idx3.shape[0]
    tok_per_w = ntok // _N_WORKERS
    mesh = plsc.VectorSubcoreMesh(core_axis_name="c", subcore_axis_name="s")

    @functools.partial(
        pl.kernel,
        out_type=jax.ShapeDtypeStruct((ntok, D_MODEL), jnp.float32),
        mesh=mesh,
        scratch_types=[
            pltpu.VMEM((tok_per_w, _N_CHUNKS * _CHUNK), jnp.int32),
            pltpu.VMEM((_CHUNK, D_MODEL), jnp.float32),
            pltpu.VMEM((_CHUNK, D_MODEL), jnp.float32),
            pltpu.VMEM((_N_CHUNKS * _CHUNK * 16,), jnp.float32),
            pltpu.VMEM((_N_CHUNKS * _CHUNK * 16,), jnp.float32),
            pltpu.VMEM((D_MODEL,), jnp.float32),
            pltpu.VMEM((D_MODEL,), jnp.float32),
            pltpu.SemaphoreType.DMA,
            pltpu.SemaphoreType.DMA,
            pltpu.SemaphoreType.DMA,
            pltpu.SemaphoreType.DMA,
            pltpu.SemaphoreType.DMA,
            pltpu.SemaphoreType.DMA,
        ],
    )
    def bag(values_hbm, idx_hbm, wspl_hbm, out_hbm,
            idx_v, rows_a, rows_b, wspl_a, wspl_b, acc_a, acc_b,
            sem_ra, sem_rb, sem_wa, sem_wb, sem_oa, sem_ob):
        wid = lax.axis_index("s") * 2 + lax.axis_index("c")
        t0 = wid * tok_per_w
        last = tok_per_w - 1
        rbufs = (rows_a, rows_b)
        rsems = (sem_ra, sem_rb)
        wbufs = (wspl_a, wspl_b)
        wsems = (sem_wa, sem_wb)
        abufs = (acc_a, acc_b)
        osems = (sem_oa, sem_ob)

        pltpu.sync_copy(idx_hbm.at[pl.ds(t0, tok_per_w)], idx_v)
        pltpu.async_copy(wspl_hbm.at[t0], wspl_a, sem_wa)
        pltpu.async_copy(wspl_hbm.at[t0 + 1], wspl_b, sem_wb)
        pltpu.async_copy(values_hbm.at[idx_v.at[0, pl.ds(0, _CHUNK)]], rows_a, sem_ra)

        def pair_body(p, _):
            for par in range(2):
                ti = 2 * p + par
                tok = t0 + ti
                wbuf = wbufs[par]
                acc = abufs[par]
                pltpu.make_async_copy(wspl_hbm.at[tok], wbuf, wsems[par]).wait()

                @pl.when(p > 0)
                def _():
                    pltpu.make_async_copy(
                        acc, out_hbm.at[tok - 2], osems[par]).wait()

                for c in range(_N_CHUNKS):
                    buf = rbufs[c % 2]
                    pltpu.make_async_copy(
                        values_hbm.at[idx_v.at[ti, pl.ds(c * _CHUNK, _CHUNK)]],
                        buf, rsems[c % 2]).wait()
                    if c + 1 < _N_CHUNKS:
                        pltpu.async_copy(
                            values_hbm.at[
                                idx_v.at[ti, pl.ds((c + 1) * _CHUNK, _CHUNK)]],
                            rbufs[(c + 1) % 2], rsems[(c + 1) % 2])
                    else:
                        tnext = jnp.minimum(ti + 1, last)
                        pltpu.async_copy(
                            values_hbm.at[idx_v.at[tnext, pl.ds(0, _CHUNK)]],
                            rbufs[0], rsems[0])
                    w = [wbuf[pl.ds((c * _CHUNK + r) * 16, 16)]
                         for r in range(_CHUNK)]

                    def d_body(dd):
                        sl = pl.ds(dd * 16, 16)
                        t = [buf.at[r][sl] * w[r] for r in range(_CHUNK)]
                        while len(t) > 1:
                            t = [t[i] + t[i + 1] for i in range(0, len(t), 2)]
                        s = t[0]
                        if c == 0:
                            acc[sl] = s
                        else:
                            plsc.addupdate(acc.at[sl], s)

                    plsc.parallel_loop(0, _DCHUNKS, 1, unroll=_D_UNROLL)(
                        lambda dd: d_body(dd))
                tpre = t0 + jnp.minimum(ti + 2, last)
                pltpu.async_copy(wspl_hbm.at[tpre], wbuf, wsems[par])
                pltpu.async_copy(acc, out_hbm.at[tok], osems[par])
            return 0

        lax.fori_loop(0, tok_per_w // 2, pair_body, 0)
        pltpu.make_async_copy(
            wspl_hbm.at[t0 + last], wspl_a, sem_wa).wait()
        pltpu.make_async_copy(
            wspl_hbm.at[t0 + last], wspl_b, sem_wb).wait()
        pltpu.make_async_copy(
            values_hbm.at[idx_v.at[last, pl.ds(0, _CHUNK)]], rows_a, sem_ra).wait()
        pltpu.make_async_copy(
            acc_a, out_hbm.at[t0 + last - 1], sem_oa).wait()
        pltpu.make_async_copy(
            acc_b, out_hbm.at[t0 + last], sem_ob).wait()

    return bag(values, idx3, wspl3)


def _back_body(gate_ref, bag_ref, wv_ref, bv_ref, out_ref):
    y = gate_ref[...] * bag_ref[...]
    out_ref[...] = jnp.dot(y, wv_ref[...], preferred_element_type=jnp.float32,
                           precision=_PREC) + bv_ref[...]


def _back_call(gate, bag, W_vproj, b_vproj):
    ntok = gate.shape[0]
    return pl.pallas_call(
        _back_body,
        grid=(ntok // TOK_BLK,),
        in_specs=[
            pl.BlockSpec((TOK_BLK, D_MODEL), lambda i: (i, 0)),
            pl.BlockSpec((TOK_BLK, D_MODEL), lambda i: (i, 0)),
            pl.BlockSpec((D_MODEL, D_MODEL), lambda i: (0, 0)),
            pl.BlockSpec((1, D_MODEL), lambda i: (0, 0)),
        ],
        out_specs=pl.BlockSpec((TOK_BLK, D_MODEL), lambda i: (i, 0)),
        out_shape=jax.ShapeDtypeStruct((ntok, D_MODEL), jnp.float32),
    )(gate, bag, W_vproj, b_vproj)


def kernel(input, W_query, b_query, keys, values, W_swilu, b_swilu, W_vproj, b_vproj):
    x = input.reshape(-1, D_MODEL)
    keys_r = keys.reshape(HEADS, 2, N_KEYS, K_DIM // 2)
    K1T = keys_r[:, 0].transpose(0, 2, 1)
    K2T = keys_r[:, 1].transpose(0, 2, 1)
    G1 = jnp.asarray(_G1_np)
    G2 = jnp.asarray(_G2_np)
    PAD = jnp.asarray(_PAD_np)
    halves = []
    nh = N_TOKENS // _N_HALVES
    for hh in range(_N_HALVES):
        xh = x[hh * nh:(hh + 1) * nh]
        gate, idx, wts = _front_call(
            xh, W_query, b_query.reshape(1, -1), K1T, K2T,
            W_swilu, b_swilu.reshape(1, -1), G1, G2, PAD)
        wspl3 = jnp.broadcast_to(
            wts[..., None], (nh, HEADS * KNN, 16)
        ).reshape(nh, HEADS * KNN * 16)
        bag = _bag_call(values, idx, wspl3)
        halves.append(_back_call(gate, bag, W_vproj, b_vproj.reshape(1, -1)))
    return jnp.concatenate(halves, axis=0)


# speedup vs baseline: 1.9035x; 1.0671x over previous
import functools

import numpy as np
import jax
import jax.numpy as jnp
from jax import lax
from jax.experimental import pallas as pl
from jax.experimental.pallas import tpu as pltpu
from jax.experimental.pallas import tpu_sc as plsc

HEADS = 4
K_DIM = 512
KNN = 32
N_KEYS = 128
D_MODEL = 2048
N_TOKENS = 4096

TOK_BLK = 256
N_BLKS = N_TOKENS // TOK_BLK

_PREC = lax.Precision.DEFAULT
_PREC_SEL = lax.Precision.HIGHEST

_pairs = [(i, j) for i in range(KNN) for j in range(KNN) if (i + 1) * (j + 1) <= KNN]
_N_CAND = 128
_G1_np = np.zeros((KNN, _N_CAND), np.float32)
_G2_np = np.zeros((KNN, _N_CAND), np.float32)
_PAD_np = np.zeros((1, _N_CAND), np.float32)
for _p, (_i, _j) in enumerate(_pairs):
    _G1_np[_i, _p] = 1.0
    _G2_np[_j, _p] = 1.0
for _p in range(len(_pairs), _N_CAND):
    _PAD_np[0, _p] = -np.inf


def _top32(s, iota_n):
    B = s.shape[0]
    iota_k = lax.broadcasted_iota(jnp.int32, (B, KNN), 1)

    def body(k, carry):
        s, sc, idc = carry
        m = jnp.max(s, axis=1, keepdims=True)
        am = jnp.min(jnp.where(s == m, iota_n, s.shape[1]), axis=1, keepdims=True)
        s = jnp.where(iota_n == am, -jnp.inf, s)
        koh = iota_k == k
        sc = jnp.where(koh, m, sc)
        idc = jnp.where(koh, am, idc)
        return s, sc, idc

    init = (s, jnp.full((B, KNN), -jnp.inf, jnp.float32), jnp.zeros((B, KNN), jnp.int32))
    _, sc, idc = lax.fori_loop(0, KNN, body, init)
    return sc, idc


def _front_body(x_ref, wq_ref, bq_ref, k1t_ref, k2t_ref, ws_ref, bs_ref,
                g1_ref, g2_ref, pad_ref, gate_ref, idx_ref, wts_ref):
    B = TOK_BLK
    half = K_DIM // 2
    x = x_ref[...]
    q = jnp.dot(x, wq_ref[...], preferred_element_type=jnp.float32,
                precision=_PREC) + bq_ref[...]
    gate_ref[...] = jax.nn.silu(
        jnp.dot(x, ws_ref[...], preferred_element_type=jnp.float32,
                precision=_PREC) + bs_ref[...])

    iota_n = lax.broadcasted_iota(jnp.int32, (B, N_KEYS), 1)
    iota_c = lax.broadcasted_iota(jnp.int32, (B, _N_CAND), 1)
    g1 = g1_ref[...]
    g2 = g2_ref[...]
    pad = pad_ref[...]

    for h in range(HEADS):
        q1 = q[:, h * K_DIM:h * K_DIM + half]
        q2 = q[:, h * K_DIM + half:(h + 1) * K_DIM]
        s1 = jnp.dot(q1, k1t_ref[h], preferred_element_type=jnp.float32,
                     precision=_PREC)
        s2 = jnp.dot(q2, k2t_ref[h], preferred_element_type=jnp.float32,
                     precision=_PREC)
        sc1, id1 = _top32(s1, iota_n)
        sc2, id2 = _top32(s2, iota_n)
        c = (jnp.dot(sc1, g1, preferred_element_type=jnp.float32, precision=_PREC_SEL)
             + jnp.dot(sc2, g2, preferred_element_type=jnp.float32, precision=_PREC_SEL)
             + pad)
        icomb = (jnp.dot(id1.astype(jnp.float32), g1,
                         preferred_element_type=jnp.float32, precision=_PREC_SEL) * N_KEYS
                 + jnp.dot(id2.astype(jnp.float32), g2,
                           preferred_element_type=jnp.float32, precision=_PREC_SEL))

        iota_k = lax.broadcasted_iota(jnp.int32, (B, KNN), 1)

        def body2(k, carry):
            c, sc, idc = carry
            m = jnp.max(c, axis=1, keepdims=True)
            am = jnp.min(jnp.where(c == m, iota_c, _N_CAND), axis=1, keepdims=True)
            c = jnp.where(iota_c == am, -jnp.inf, c)
            iv = jnp.sum(jnp.where(iota_c == am, icomb, 0.0), axis=1, keepdims=True)
            koh = iota_k == k
            sc = jnp.where(koh, m, sc)
            idc = jnp.where(koh, iv, idc)
            return c, sc, idc

        init = (c, jnp.full((B, KNN), -jnp.inf, jnp.float32),
                jnp.zeros((B, KNN), jnp.float32))
        _, sc, idc = lax.fori_loop(0, KNN, body2, init)

        e = jnp.exp(sc - sc[:, 0:1])
        w = e / jnp.sum(e, axis=1, keepdims=True)
        idx_ref[:, h * KNN:(h + 1) * KNN] = idc.astype(jnp.int32)
        wts_ref[:, h * KNN:(h + 1) * KNN] = w


def _front_call(x, W_query, b_query, K1T, K2T, W_swilu, b_swilu, G1, G2, PAD):
    ntok = x.shape[0]
    full = lambda shape: pl.BlockSpec(shape, lambda i: tuple(0 for _ in shape))
    return pl.pallas_call(
        _front_body,
        grid=(ntok // TOK_BLK,),
        in_specs=[
            pl.BlockSpec((TOK_BLK, D_MODEL), lambda i: (i, 0)),
            full((D_MODEL, HEADS * K_DIM)),
            full((1, HEADS * K_DIM)),
            full((HEADS, K_DIM // 2, N_KEYS)),
            full((HEADS, K_DIM // 2, N_KEYS)),
            full((D_MODEL, D_MODEL)),
            full((1, D_MODEL)),
            full((KNN, _N_CAND)),
            full((KNN, _N_CAND)),
            full((1, _N_CAND)),
        ],
        out_specs=[
            pl.BlockSpec((TOK_BLK, D_MODEL), lambda i: (i, 0)),
            pl.BlockSpec((TOK_BLK, HEADS * KNN), lambda i: (i, 0)),
            pl.BlockSpec((TOK_BLK, HEADS * KNN), lambda i: (i, 0)),
        ],
        out_shape=[
            jax.ShapeDtypeStruct((ntok, D_MODEL), jnp.float32),
            jax.ShapeDtypeStruct((ntok, HEADS * KNN), jnp.int32),
            jax.ShapeDtypeStruct((ntok, HEADS * KNN), jnp.float32),
        ],
    )(x, W_query, b_query, K1T, K2T, W_swilu, b_swilu, G1, G2, PAD)


_N_WORKERS = 32
_TOK_PER_W = N_TOKENS // _N_WORKERS
_CHUNK = 16
_N_CHUNKS = (HEADS * KNN) // _CHUNK
_DCHUNKS = D_MODEL // 16


_D_UNROLL = 4
_N_HALVES = 8
_NWORDS = D_MODEL // 2


def _bag_call(values, idx3, wspl3):
    ntok = idx3.shape[0]
    tok_per_w = ntok // _N_WORKERS
    mesh = plsc.VectorSubcoreMesh(core_axis_name="c", subcore_axis_name="s")

    @functools.partial(
        pl.kernel,
        out_type=jax.ShapeDtypeStruct((ntok, D_MODEL), jnp.float32),
        mesh=mesh,
        scratch_types=[
            pltpu.VMEM((tok_per_w, _N_CHUNKS * _CHUNK), jnp.int32),
            pltpu.VMEM((_CHUNK, D_MODEL), jnp.float32),
            pltpu.VMEM((_CHUNK, D_MODEL), jnp.float32),
            pltpu.VMEM((_N_CHUNKS * _CHUNK * 16,), jnp.float32),
            pltpu.VMEM((_N_CHUNKS * _CHUNK * 16,), jnp.float32),
            pltpu.VMEM((D_MODEL,), jnp.float32),
            pltpu.VMEM((D_MODEL,), jnp.float32),
            pltpu.SemaphoreType.DMA,
            pltpu.SemaphoreType.DMA,
            pltpu.SemaphoreType.DMA,
            pltpu.SemaphoreType.DMA,
            pltpu.SemaphoreType.DMA,
            pltpu.SemaphoreType.DMA,
        ],
    )
    def bag(values_hbm, idx_hbm, wspl_hbm, out_hbm,
            idx_v, rows_a, rows_b, wspl_a, wspl_b, acc_a, acc_b,
            sem_ra, sem_rb, sem_wa, sem_wb, sem_oa, sem_ob):
        wid = lax.axis_index("s") * 2 + lax.axis_index("c")
        t0 = wid * tok_per_w
        last = tok_per_w - 1
        rbufs = (rows_a, rows_b)
        rsems = (sem_ra, sem_rb)
        wbufs = (wspl_a, wspl_b)
        wsems = (sem_wa, sem_wb)
        abufs = (acc_a, acc_b)
        osems = (sem_oa, sem_ob)

        pltpu.sync_copy(idx_hbm.at[pl.ds(t0, tok_per_w)], idx_v)
        pltpu.async_copy(wspl_hbm.at[t0], wspl_a, sem_wa)
        pltpu.async_copy(wspl_hbm.at[t0 + 1], wspl_b, sem_wb)
        pltpu.async_copy(values_hbm.at[idx_v.at[0, pl.ds(0, _CHUNK)]], rows_a, sem_ra)

        def pair_body(p, _):
            for par in range(2):
                ti = 2 * p + par
                tok = t0 + ti
                wbuf = wbufs[par]
                acc = abufs[par]
                pltpu.make_async_copy(wspl_hbm.at[tok], wbuf, wsems[par]).wait()

                @pl.when(p > 0)
                def _():
                    pltpu.make_async_copy(
                        acc, out_hbm.at[tok - 2], osems[par]).wait()

                for c in range(_N_CHUNKS):
                    buf = rbufs[c % 2]
                    pltpu.make_async_copy(
                        values_hbm.at[idx_v.at[ti, pl.ds(c * _CHUNK, _CHUNK)]],
                        buf, rsems[c % 2]).wait()
                    if c + 1 < _N_CHUNKS:
                        pltpu.async_copy(
                            values_hbm.at[
                                idx_v.at[ti, pl.ds((c + 1) * _CHUNK, _CHUNK)]],
                            rbufs[(c + 1) % 2], rsems[(c + 1) % 2])
                    else:
                        tnext = jnp.minimum(ti + 1, last)
                        pltpu.async_copy(
                            values_hbm.at[idx_v.at[tnext, pl.ds(0, _CHUNK)]],
                            rbufs[0], rsems[0])
                    w = [wbuf[pl.ds((c * _CHUNK + r) * 16, 16)]
                         for r in range(_CHUNK)]

                    def d_body(dd):
                        sl = pl.ds(dd * 16, 16)
                        t = [buf.at[r][sl] * w[r] for r in range(_CHUNK)]
                        while len(t) > 1:
                            t = [t[i] + t[i + 1] for i in range(0, len(t), 2)]
                        s = t[0]
                        if c == 0:
                            acc[sl] = s
                        else:
                            plsc.addupdate(acc.at[sl], s)

                    plsc.parallel_loop(0, _DCHUNKS, 1, unroll=_D_UNROLL)(
                        lambda dd: d_body(dd))
                tpre = t0 + jnp.minimum(ti + 2, last)
                pltpu.async_copy(wspl_hbm.at[tpre], wbuf, wsems[par])
                pltpu.async_copy(acc, out_hbm.at[tok], osems[par])
            return 0

        lax.fori_loop(0, tok_per_w // 2, pair_body, 0)
        pltpu.make_async_copy(
            wspl_hbm.at[t0 + last], wspl_a, sem_wa).wait()
        pltpu.make_async_copy(
            wspl_hbm.at[t0 + last], wspl_b, sem_wb).wait()
        pltpu.make_async_copy(
            values_hbm.at[idx_v.at[last, pl.ds(0, _CHUNK)]], rows_a, sem_ra).wait()
        pltpu.make_async_copy(
            acc_a, out_hbm.at[t0 + last - 1], sem_oa).wait()
        pltpu.make_async_copy(
            acc_b, out_hbm.at[t0 + last], sem_ob).wait()

    return bag(values, idx3, wspl3)


def _back_body(gate_ref, bag_ref, wv_ref, bv_ref, out_ref):
    y = gate_ref[...] * bag_ref[...]
    out_ref[...] = jnp.dot(y, wv_ref[...], preferred_element_type=jnp.float32,
                           precision=_PREC) + bv_ref[...]


def _back_call(gate, bag, W_vproj, b_vproj):
    ntok = gate.shape[0]
    return pl.pallas_call(
        _back_body,
        grid=(ntok // TOK_BLK,),
        in_specs=[
            pl.BlockSpec((TOK_BLK, D_MODEL), lambda i: (i, 0)),
            pl.BlockSpec((TOK_BLK, D_MODEL), lambda i: (i, 0)),
            pl.BlockSpec((D_MODEL, D_MODEL), lambda i: (0, 0)),
            pl.BlockSpec((1, D_MODEL), lambda i: (0, 0)),
        ],
        out_specs=pl.BlockSpec((TOK_BLK, D_MODEL), lambda i: (i, 0)),
        out_shape=jax.ShapeDtypeStruct((ntok, D_MODEL), jnp.float32),
    )(gate, bag, W_vproj, b_vproj)


def kernel(input, W_query, b_query, keys, values, W_swilu, b_swilu, W_vproj, b_vproj):
    x = input.reshape(-1, D_MODEL)
    keys_r = keys.reshape(HEADS, 2, N_KEYS, K_DIM // 2)
    K1T = keys_r[:, 0].transpose(0, 2, 1)
    K2T = keys_r[:, 1].transpose(0, 2, 1)
    G1 = jnp.asarray(_G1_np)
    G2 = jnp.asarray(_G2_np)
    PAD = jnp.asarray(_PAD_np)
    halves = []
    nh = N_TOKENS // _N_HALVES
    for hh in range(_N_HALVES):
        xh = x[hh * nh:(hh + 1) * nh]
        gate, idx, wts = _front_call(
            xh, W_query, b_query.reshape(1, -1), K1T, K2T,
            W_swilu, b_swilu.reshape(1, -1), G1, G2, PAD)
        wspl3 = jnp.broadcast_to(
            wts[..., None], (nh, HEADS * KNN, 16)
        ).reshape(nh, HEADS * KNN * 16)
        bag = _bag_call(values, idx, wspl3)
        halves.append(_back_call(gate, bag, W_vproj, b_vproj.reshape(1, -1)))
    return jnp.concatenate(halves, axis=0)
